# baseline (speedup 1.0000x reference)
import functools

import jax
import jax.numpy as jnp
from jax import lax
from jax.experimental import pallas as pl
from jax.experimental.pallas import tpu as pltpu
from jax.experimental.pallas import tpu_sc as plsc

_NC = 2
_NS = 16
_NW = _NC * _NS

_CHUNK = 128
_LOOKAHEAD = 4
_NBUF = 8


@jax.jit
def _embed_lookup(x, embedding):
    NI, NJ = x.shape
    V, D = embedding.shape
    B = NI * NJ
    b_per_w = B // _NW
    n_chunks = b_per_w // _CHUNK
    assert b_per_w * _NW == B and n_chunks * _CHUNK == b_per_w
    assert n_chunks > _NBUF

    table2 = lax.optimization_barrier(embedding.reshape(V // 2, 2 * D))
    table_lin = table2.reshape(V, D)
    idx3 = x.reshape(_NW, n_chunks, _CHUNK)

    mesh = plsc.VectorSubcoreMesh(core_axis_name="c", subcore_axis_name="s")

    @functools.partial(
        pl.kernel,
        mesh=mesh,
        out_type=jax.ShapeDtypeStruct((B, D), jnp.float32),
        scratch_types=[
            pltpu.VMEM((n_chunks, _CHUNK), jnp.int32),
            pltpu.VMEM((_NBUF, _CHUNK, D), jnp.float32),
            pltpu.SemaphoreType.DMA,
            pltpu.SemaphoreType.DMA,
        ],
        compiler_params=pltpu.CompilerParams(
            use_tc_tiling_on_sc=False, needs_layout_passes=False
        ),
    )
    def gather_kernel(idx_hbm, table_hbm, out_hbm, idx_v, rows_v, gsem, wsem):
        wid = lax.axis_index("s") * _NC + lax.axis_index("c")
        base = wid * b_per_w
        pltpu.sync_copy(idx_hbm.at[wid], idx_v)

        def start_gather(j):
            pltpu.async_copy(
                table_hbm.at[idx_v.at[j]], rows_v.at[j % _NBUF], gsem
            )

        def start_write(j):
            pltpu.async_copy(
                rows_v.at[j % _NBUF],
                out_hbm.at[pl.ds(base + j * _CHUNK, _CHUNK)],
                wsem,
            )

        def wait_gather_one():
            pltpu.make_async_copy(
                table_hbm.at[idx_v.at[0]], rows_v.at[0], gsem
            ).wait()

        def wait_write_one():
            pltpu.make_async_copy(
                rows_v.at[0], out_hbm.at[pl.ds(base, _CHUNK)], wsem
            ).wait()

        for b in range(_LOOKAHEAD):
            start_gather(b)

        @pl.loop(0, n_chunks)
        def _chunk(j):
            jn = j + _LOOKAHEAD

            @pl.when(jn < n_chunks)
            def _():
                @pl.when(jn >= _NBUF)
                def _():
                    wait_write_one()

                start_gather(jn)

            wait_gather_one()
            start_write(j)

        for _ in range(_NBUF):
            wait_write_one()

    out64 = gather_kernel(idx3, table_lin)
    out2 = lax.optimization_barrier(out64.reshape(B // 2, 2 * D))
    return out2.reshape(NI, NJ, D)


def kernel(x, embedding):
    return (_embed_lookup(x, embedding), None)


# speedup vs baseline: 1.3269x; 1.3269x over previous
import functools

import jax
import jax.numpy as jnp
from jax import lax
from jax.experimental import pallas as pl
from jax.experimental.pallas import tpu as pltpu
from jax.experimental.pallas import tpu_sc as plsc

_NC = 2
_NS = 16
_NW = _NC * _NS

_CHUNK = 128
_LOOKAHEAD = 4
_NBUF = 8


@jax.jit
def _embed_lookup(x, embedding):
    NI, NJ = x.shape
    V, D = embedding.shape
    B = NI * NJ
    b_per_w = B // _NW
    n_chunks = b_per_w // _CHUNK
    assert b_per_w * _NW == B and n_chunks * _CHUNK == b_per_w
    assert n_chunks > _NBUF

    table2 = lax.optimization_barrier(embedding.reshape(V // 2, 2 * D))
    table_lin = table2.reshape(V, D)
    idx3 = x.reshape(_NW, n_chunks, _CHUNK)

    mesh = plsc.VectorSubcoreMesh(core_axis_name="c", subcore_axis_name="s")

    @functools.partial(
        pl.kernel,
        mesh=mesh,
        out_type=jax.ShapeDtypeStruct((B, 2 * D), jnp.float32),
        scratch_types=[
            pltpu.VMEM((n_chunks, _CHUNK), jnp.int32),
            pltpu.VMEM((_NBUF, _CHUNK, D), jnp.float32),
            pltpu.SemaphoreType.DMA,
            pltpu.SemaphoreType.DMA,
        ],
        compiler_params=pltpu.CompilerParams(
            use_tc_tiling_on_sc=False, needs_layout_passes=False
        ),
    )
    def gather_kernel(idx_hbm, table_hbm, out_hbm, idx_v, rows_v, gsem, wsem):
        wid = lax.axis_index("s") * _NC + lax.axis_index("c")
        base = wid * b_per_w
        pltpu.sync_copy(idx_hbm.at[wid], idx_v)

        def start_gather(j):
            pltpu.async_copy(
                table_hbm.at[idx_v.at[j]], rows_v.at[j % _NBUF], gsem
            )

        def start_write(j):
            pltpu.async_copy(
                rows_v.at[j % _NBUF],
                out_hbm.at[pl.ds(base + j * _CHUNK, _CHUNK), pl.ds(0, D)],
                wsem,
            )

        def wait_gather_one():
            pltpu.make_async_copy(
                table_hbm.at[idx_v.at[0]], rows_v.at[0], gsem
            ).wait()

        def wait_write_one():
            pltpu.make_async_copy(
                rows_v.at[0],
                out_hbm.at[pl.ds(base, _CHUNK), pl.ds(0, D)],
                wsem,
            ).wait()

        for b in range(_LOOKAHEAD):
            start_gather(b)

        @pl.loop(0, n_chunks)
        def _chunk(j):
            jn = j + _LOOKAHEAD

            @pl.when(jn < n_chunks)
            def _():
                @pl.when(jn >= _NBUF)
                def _():
                    wait_write_one()

                start_gather(jn)

            wait_gather_one()
            start_write(j)

        for _ in range(_NBUF):
            wait_write_one()

    out128 = gather_kernel(idx3, table_lin)
    return out128[:, :D].reshape(NI, NJ, D)


def kernel(x, embedding):
    return (_embed_lookup(x, embedding), None)
